# baseline (speedup 1.0000x reference)
import jax
import jax.numpy as jnp
from jax.experimental import pallas as pl

_ROWS = 4096
_COLS = 16384
_TOTAL = _ROWS * _COLS
_BLOCK = 2 * 1024 * 1024


def _copy_block(in_ref, out_ref):
    out_ref[...] = in_ref[...]


def kernel(x):
    flat = jnp.reshape(x, (_TOTAL,))
    out = pl.pallas_call(
        _copy_block,
        grid=(_TOTAL // _BLOCK,),
        in_specs=[pl.BlockSpec((_BLOCK,), lambda i: (i,))],
        out_specs=pl.BlockSpec((_BLOCK,), lambda i: (i,)),
        out_shape=jax.ShapeDtypeStruct((_TOTAL,), x.dtype),
    )(flat)
    return jnp.reshape(out, (_ROWS, _COLS))


# speedup vs baseline: 3.8858x; 3.8858x over previous
import jax
import jax.numpy as jnp
from jax.experimental import pallas as pl

_ROWS = 4096
_COLS = 16384
_BLOCK_ROWS = 128


def _copy_block(in_ref, out_ref):
    out_ref[...] = in_ref[...]


def kernel(x):
    grid = (_ROWS // _BLOCK_ROWS,)
    return pl.pallas_call(
        _copy_block,
        grid=grid,
        in_specs=[pl.BlockSpec((_BLOCK_ROWS, _COLS), lambda i: (i, 0))],
        out_specs=pl.BlockSpec((_BLOCK_ROWS, _COLS), lambda i: (i, 0)),
        out_shape=jax.ShapeDtypeStruct((_ROWS, _COLS), x.dtype),
    )(x)


# speedup vs baseline: 3.8870x; 1.0003x over previous
import functools
import jax
import jax.numpy as jnp
from jax.experimental import pallas as pl
from jax.experimental.pallas import tpu as pltpu

_ROWS = 4096
_COLS = 16384
_BLOCK_ROWS = 128

_tc_mesh = pltpu.create_tensorcore_mesh("t", num_cores=1)


@functools.partial(
    pl.kernel,
    out_type=jax.ShapeDtypeStruct((_ROWS, _COLS), jnp.float32),
    mesh=_tc_mesh,
)
def _copy(x_hbm, out_hbm):
    def copy_block(in_ref, out_ref):
        out_ref[...] = in_ref[...]

    pltpu.emit_pipeline(
        copy_block,
        grid=(_ROWS // _BLOCK_ROWS,),
        in_specs=[pl.BlockSpec((_BLOCK_ROWS, _COLS), lambda i: (i, 0))],
        out_specs=[pl.BlockSpec((_BLOCK_ROWS, _COLS), lambda i: (i, 0))],
    )(x_hbm, out_hbm)


def kernel(x):
    return _copy(x)
